# baseline (device time: 67671 ns/iter reference)
import jax
import jax.numpy as jnp
from jax import lax
from jax.experimental import pallas as pl
from jax.experimental.pallas import tpu as pltpu


def kernel(O, Wo):
    B, S, H, D = O.shape
    K = H * D
    N = Wo.shape[1]
    X = 2
    S_out = S // X

    O2 = O.reshape(B, S, K)

    def body(o_ref, wo_ref, out_ref, wo_bf, send_buf, recv_buf,
             send_sem, recv_sem):
        my_x = lax.axis_index("x")
        my_y = lax.axis_index("y")
        peer_x = 1 - my_x

        barrier = pltpu.get_barrier_semaphore()
        pl.semaphore_signal(
            barrier, inc=1,
            device_id=(peer_x, my_y),
            device_id_type=pl.DeviceIdType.MESH,
        )
        pl.semaphore_wait(barrier, 1)

        wo_bf[...] = wo_ref[...].astype(jnp.bfloat16)

        for b in range(B):
            a = o_ref[b, pl.ds(peer_x * S_out, S_out), :].astype(jnp.bfloat16)
            send_buf[b, :, :] = lax.dot(
                a, wo_bf[...], preferred_element_type=jnp.float32
            ).astype(jnp.bfloat16)

        rdma = pltpu.make_async_remote_copy(
            src_ref=send_buf,
            dst_ref=recv_buf,
            send_sem=send_sem,
            recv_sem=recv_sem,
            device_id=(peer_x, my_y),
            device_id_type=pl.DeviceIdType.MESH,
        )
        rdma.start()

        for b in range(B):
            a = o_ref[b, pl.ds(my_x * S_out, S_out), :].astype(jnp.bfloat16)
            out_ref[b, :, :] = lax.dot(
                a, wo_bf[...], preferred_element_type=jnp.float32
            )

        rdma.wait()

        for b in range(B):
            out_ref[b, :, :] = (
                out_ref[b, :, :] + recv_buf[b, :, :].astype(jnp.float32)
            )

    return pl.pallas_call(
        body,
        out_shape=jax.ShapeDtypeStruct((B, S_out, N), jnp.float32),
        in_specs=[
            pl.BlockSpec(memory_space=pltpu.VMEM),
            pl.BlockSpec(memory_space=pltpu.VMEM),
        ],
        out_specs=pl.BlockSpec(memory_space=pltpu.VMEM),
        scratch_shapes=[
            pltpu.VMEM((K, N), jnp.bfloat16),
            pltpu.VMEM((B, S_out, N), jnp.bfloat16),
            pltpu.VMEM((B, S_out, N), jnp.bfloat16),
            pltpu.SemaphoreType.DMA,
            pltpu.SemaphoreType.DMA,
        ],
        compiler_params=pltpu.CompilerParams(collective_id=0),
    )(O2, Wo)


# device time: 63710 ns/iter; 1.0622x vs baseline; 1.0622x over previous
import jax
import jax.numpy as jnp
from jax import lax
from jax.experimental import pallas as pl
from jax.experimental.pallas import tpu as pltpu

NCHUNK = 4


def kernel(O, Wo):
    B, S, H, D = O.shape
    K = H * D
    N = Wo.shape[1]
    X = 2
    S_out = S // X
    SPLIT = NCHUNK // B
    R = S_out // SPLIT

    O2 = O.reshape(B, S, K)

    def body(o_ref, wo_ref, out_ref, wo_bf, send_buf, recv_buf,
             send_sems, recv_sems):
        my_x = lax.axis_index("x")
        my_y = lax.axis_index("y")
        peer_x = 1 - my_x

        barrier = pltpu.get_barrier_semaphore()
        pl.semaphore_signal(
            barrier, inc=1,
            device_id=(peer_x, my_y),
            device_id_type=pl.DeviceIdType.MESH,
        )
        pl.semaphore_wait(barrier, 1)

        wo_bf[...] = wo_ref[...].astype(jnp.bfloat16)

        def rows(c, x):
            return pl.ds(x * S_out + (c % SPLIT) * R, R)

        rdmas = []
        for c in range(NCHUNK):
            a = o_ref[c // SPLIT, rows(c, peer_x), :].astype(jnp.bfloat16)
            send_buf[c, :, :] = lax.dot(
                a, wo_bf[...], preferred_element_type=jnp.float32
            ).astype(jnp.bfloat16)
            rdma = pltpu.make_async_remote_copy(
                src_ref=send_buf.at[c],
                dst_ref=recv_buf.at[c],
                send_sem=send_sems.at[c],
                recv_sem=recv_sems.at[c],
                device_id=(peer_x, my_y),
                device_id_type=pl.DeviceIdType.MESH,
            )
            rdma.start()
            rdmas.append(rdma)

        for c in range(NCHUNK):
            a = o_ref[c // SPLIT, rows(c, my_x), :].astype(jnp.bfloat16)
            out_ref[c // SPLIT, (c % SPLIT) * R:(c % SPLIT + 1) * R, :] = (
                lax.dot(a, wo_bf[...], preferred_element_type=jnp.float32)
            )

        for c in range(NCHUNK):
            rdmas[c].wait_recv()
            sl = pl.ds((c % SPLIT) * R, R)
            out_ref[c // SPLIT, sl, :] = (
                out_ref[c // SPLIT, sl, :]
                + recv_buf[c, :, :].astype(jnp.float32)
            )

        for c in range(NCHUNK):
            rdmas[c].wait_send()

    return pl.pallas_call(
        body,
        out_shape=jax.ShapeDtypeStruct((B, S_out, N), jnp.float32),
        in_specs=[
            pl.BlockSpec(memory_space=pltpu.VMEM),
            pl.BlockSpec(memory_space=pltpu.VMEM),
        ],
        out_specs=pl.BlockSpec(memory_space=pltpu.VMEM),
        scratch_shapes=[
            pltpu.VMEM((K, N), jnp.bfloat16),
            pltpu.VMEM((NCHUNK, R, N), jnp.bfloat16),
            pltpu.VMEM((NCHUNK, R, N), jnp.bfloat16),
            pltpu.SemaphoreType.DMA((NCHUNK,)),
            pltpu.SemaphoreType.DMA((NCHUNK,)),
        ],
        compiler_params=pltpu.CompilerParams(collective_id=0),
    )(O2, Wo)


# device time: 48887 ns/iter; 1.3842x vs baseline; 1.3032x over previous
import jax
import jax.numpy as jnp
from jax import lax
from jax.experimental import pallas as pl
from jax.experimental.pallas import tpu as pltpu

NCHUNK = 4


def kernel(O, Wo):
    B, S, H, D = O.shape
    K = H * D
    N = Wo.shape[1]
    S_out = S // 2
    Q = S_out // 2
    SPLIT = NCHUNK // B
    R = Q // SPLIT

    O2 = O.reshape(B, S, K)

    def body(o_ref, wo_ref, out_ref, wo_bf, x_send, x_recv, y_send, y_recv,
             x_send_sems, x_recv_sems, y_send_sems, y_recv_sems):
        my_x = lax.axis_index("x")
        my_y = lax.axis_index("y")
        peer_x = 1 - my_x
        peer_y = 1 - my_y

        barrier = pltpu.get_barrier_semaphore()
        pl.semaphore_signal(
            barrier, inc=1, device_id=(peer_x, my_y),
            device_id_type=pl.DeviceIdType.MESH,
        )
        pl.semaphore_signal(
            barrier, inc=1, device_id=(my_x, peer_y),
            device_id_type=pl.DeviceIdType.MESH,
        )
        pl.semaphore_wait(barrier, 2)

        wo_bf[...] = wo_ref[...].astype(jnp.bfloat16)

        def chunk(c):
            return c // SPLIT, (c % SPLIT) * R

        x_rdmas = []
        for c in range(NCHUNK):
            b, r = chunk(c)
            a = o_ref[b, pl.ds(peer_x * S_out + my_y * Q + r, R), :]
            x_send[c, :, :] = lax.dot(
                a.astype(jnp.bfloat16), wo_bf[...],
                preferred_element_type=jnp.float32,
            ).astype(jnp.bfloat16)
            rdma = pltpu.make_async_remote_copy(
                src_ref=x_send.at[c],
                dst_ref=x_recv.at[c],
                send_sem=x_send_sems.at[c],
                recv_sem=x_recv_sems.at[c],
                device_id=(peer_x, my_y),
                device_id_type=pl.DeviceIdType.MESH,
            )
            rdma.start()
            x_rdmas.append(rdma)

        for c in range(NCHUNK):
            b, r = chunk(c)
            a = o_ref[b, pl.ds(my_x * S_out + my_y * Q + r, R), :]
            out_ref[b, pl.ds(my_y * Q + r, R), :] = lax.dot(
                a.astype(jnp.bfloat16), wo_bf[...],
                preferred_element_type=jnp.float32,
            )

        y_rdmas = []
        for c in range(NCHUNK):
            b, r = chunk(c)
            x_rdmas[c].wait_recv()
            sl = pl.ds(my_y * Q + r, R)
            s = out_ref[b, sl, :] + x_recv[c, :, :].astype(jnp.float32)
            out_ref[b, sl, :] = s
            y_send[c, :, :] = s.astype(jnp.bfloat16)
            rdma = pltpu.make_async_remote_copy(
                src_ref=y_send.at[c],
                dst_ref=y_recv.at[c],
                send_sem=y_send_sems.at[c],
                recv_sem=y_recv_sems.at[c],
                device_id=(my_x, peer_y),
                device_id_type=pl.DeviceIdType.MESH,
            )
            rdma.start()
            y_rdmas.append(rdma)

        for c in range(NCHUNK):
            b, r = chunk(c)
            y_rdmas[c].wait_recv()
            out_ref[b, pl.ds(peer_y * Q + r, R), :] = (
                y_recv[c, :, :].astype(jnp.float32)
            )

        for c in range(NCHUNK):
            x_rdmas[c].wait_send()
            y_rdmas[c].wait_send()

    return pl.pallas_call(
        body,
        out_shape=jax.ShapeDtypeStruct((B, S_out, N), jnp.float32),
        in_specs=[
            pl.BlockSpec(memory_space=pltpu.VMEM),
            pl.BlockSpec(memory_space=pltpu.VMEM),
        ],
        out_specs=pl.BlockSpec(memory_space=pltpu.VMEM),
        scratch_shapes=[
            pltpu.VMEM((K, N), jnp.bfloat16),
            pltpu.VMEM((NCHUNK, R, N), jnp.bfloat16),
            pltpu.VMEM((NCHUNK, R, N), jnp.bfloat16),
            pltpu.VMEM((NCHUNK, R, N), jnp.bfloat16),
            pltpu.VMEM((NCHUNK, R, N), jnp.bfloat16),
            pltpu.SemaphoreType.DMA((NCHUNK,)),
            pltpu.SemaphoreType.DMA((NCHUNK,)),
            pltpu.SemaphoreType.DMA((NCHUNK,)),
            pltpu.SemaphoreType.DMA((NCHUNK,)),
        ],
        compiler_params=pltpu.CompilerParams(collective_id=0),
    )(O2, Wo)


# device time: 45604 ns/iter; 1.4839x vs baseline; 1.0720x over previous
import jax
import jax.numpy as jnp
from jax import lax
from jax.experimental import pallas as pl
from jax.experimental.pallas import tpu as pltpu

NCHUNK = 4


def kernel(O, Wo):
    B, S, H, D = O.shape
    K = H * D
    N = Wo.shape[1]
    S_out = S // 2
    Q = S_out // 2
    SPLIT = NCHUNK // B
    R = Q // SPLIT

    OT = O.transpose(0, 2, 3, 1).reshape(B, K, S)

    def body(o_ref, wo_ref, out_ref, wo_bf, x_send, x_recv, y_send, y_recv,
             x_send_sems, x_recv_sems, y_send_sems, y_recv_sems):
        my_x = lax.axis_index("x")
        my_y = lax.axis_index("y")
        peer_x = 1 - my_x
        peer_y = 1 - my_y

        barrier = pltpu.get_barrier_semaphore()
        pl.semaphore_signal(
            barrier, inc=1, device_id=(peer_x, my_y),
            device_id_type=pl.DeviceIdType.MESH,
        )
        pl.semaphore_signal(
            barrier, inc=1, device_id=(my_x, peer_y),
            device_id_type=pl.DeviceIdType.MESH,
        )
        pl.semaphore_wait(barrier, 2)

        wo_bf[...] = wo_ref[...].astype(jnp.bfloat16)

        def chunk(c):
            return c // SPLIT, (c % SPLIT) * R

        def dot_t(a_t):
            return lax.dot_general(
                a_t.astype(jnp.bfloat16), wo_bf[...],
                dimension_numbers=(((0,), (0,)), ((), ())),
                preferred_element_type=jnp.float32,
            )

        x_rdmas = []
        for c in range(NCHUNK):
            b, r = chunk(c)
            x_send[c, :, :] = dot_t(
                o_ref[b, :, pl.ds(peer_x * S_out + my_y * Q + r, R)]
            ).astype(jnp.bfloat16)
            rdma = pltpu.make_async_remote_copy(
                src_ref=x_send.at[c],
                dst_ref=x_recv.at[c],
                send_sem=x_send_sems.at[c],
                recv_sem=x_recv_sems.at[c],
                device_id=(peer_x, my_y),
                device_id_type=pl.DeviceIdType.MESH,
            )
            rdma.start()
            x_rdmas.append(rdma)

        for c in range(NCHUNK):
            b, r = chunk(c)
            out_ref[b, pl.ds(my_y * Q + r, R), :] = dot_t(
                o_ref[b, :, pl.ds(my_x * S_out + my_y * Q + r, R)]
            )

        y_rdmas = []
        for c in range(NCHUNK):
            b, r = chunk(c)
            x_rdmas[c].wait_recv()
            sl = pl.ds(my_y * Q + r, R)
            s = out_ref[b, sl, :] + x_recv[c, :, :].astype(jnp.float32)
            out_ref[b, sl, :] = s
            y_send[c, :, :] = s.astype(jnp.bfloat16)
            rdma = pltpu.make_async_remote_copy(
                src_ref=y_send.at[c],
                dst_ref=y_recv.at[c],
                send_sem=y_send_sems.at[c],
                recv_sem=y_recv_sems.at[c],
                device_id=(my_x, peer_y),
                device_id_type=pl.DeviceIdType.MESH,
            )
            rdma.start()
            y_rdmas.append(rdma)

        for c in range(NCHUNK):
            b, r = chunk(c)
            y_rdmas[c].wait_recv()
            out_ref[b, pl.ds(peer_y * Q + r, R), :] = (
                y_recv[c, :, :].astype(jnp.float32)
            )

        for c in range(NCHUNK):
            x_rdmas[c].wait_send()
            y_rdmas[c].wait_send()

    return pl.pallas_call(
        body,
        out_shape=jax.ShapeDtypeStruct((B, S_out, N), jnp.float32),
        in_specs=[
            pl.BlockSpec(memory_space=pltpu.VMEM),
            pl.BlockSpec(memory_space=pltpu.VMEM),
        ],
        out_specs=pl.BlockSpec(memory_space=pltpu.VMEM),
        scratch_shapes=[
            pltpu.VMEM((K, N), jnp.bfloat16),
            pltpu.VMEM((NCHUNK, R, N), jnp.bfloat16),
            pltpu.VMEM((NCHUNK, R, N), jnp.bfloat16),
            pltpu.VMEM((NCHUNK, R, N), jnp.bfloat16),
            pltpu.VMEM((NCHUNK, R, N), jnp.bfloat16),
            pltpu.SemaphoreType.DMA((NCHUNK,)),
            pltpu.SemaphoreType.DMA((NCHUNK,)),
            pltpu.SemaphoreType.DMA((NCHUNK,)),
            pltpu.SemaphoreType.DMA((NCHUNK,)),
        ],
        compiler_params=pltpu.CompilerParams(collective_id=0),
    )(OT, Wo)
